# baseline (speedup 1.0000x reference)
import functools

import jax
import jax.numpy as jnp
import numpy as np
from jax import lax
from jax.experimental import pallas as pl
from jax.experimental.pallas import tpu as pltpu
from jax.experimental.pallas import tpu_sc as plsc

N_NODES = 10000
D = 128
N_EDGES = 320000

NUM_CORES = 2
NUM_SUBCORES = 16
NUM_WORKERS = NUM_CORES * NUM_SUBCORES
CHUNK = 96
SUPER = 16
SUP0 = 8
SUP1 = 6
N_BLOCKS = NUM_SUBCORES * (SUP0 + SUP1)
E_PAD = N_BLOCKS * SUPER * CHUNK
ROWS_PER_TILE = 624
ROWS_TAIL = N_NODES - NUM_SUBCORES * ROWS_PER_TILE

_Q = np.arange(D).reshape(D // 32, 2, 16).transpose(0, 2, 1).reshape(D)


def _sc_aggregate(xh, col3, row3, ew3, zeros):
    mesh = plsc.VectorSubcoreMesh(core_axis_name="c", subcore_axis_name="s")

    @functools.partial(
        pl.kernel,
        mesh=mesh,
        out_type=jax.ShapeDtypeStruct((NUM_CORES, N_NODES, D), jnp.float32),
        compiler_params=pltpu.CompilerParams(
            needs_layout_passes=False, use_tc_tiling_on_sc=False),
        scratch_types=[
            pltpu.VMEM((2, SUPER, CHUNK), jnp.int32),
            pltpu.VMEM((2, SUPER, CHUNK), jnp.int32),
            pltpu.VMEM((2, SUPER, CHUNK), jnp.float32),
            pltpu.VMEM((CHUNK, D // 2), jnp.int32),
            pltpu.VMEM((CHUNK, D // 2), jnp.int32),
            pltpu.VMEM((CHUNK, D), jnp.float32),
            pltpu.VMEM((CHUNK, D), jnp.float32),
            pltpu.VMEM_SHARED((N_NODES, D), jnp.float32),
            pltpu.SemaphoreType.DMA,
            pltpu.SemaphoreType.DMA,
            pltpu.SemaphoreType.DMA,
            pltpu.SemaphoreType.DMA,
            pltpu.SemaphoreType.DMA,
        ],
    )
    def agg(x_hbm, col_hbm, row_hbm, ew_hbm, zeros_hbm, out_hbm,
            col_s, row_s, ew_s, ga, gb, sa, sb, acc,
            g0, g1, s0, s1, isem):
        c = lax.axis_index("c")
        s = lax.axis_index("s")
        gbufs = (ga, gb)
        sbufs = (sa, sb)
        gsems = (g0, g1)
        ssems = (s0, s1)

        nsup = jnp.where(c == 0, jnp.int32(SUP0), jnp.int32(SUP1))
        blk_base = jnp.where(c == 0, s * SUP0,
                             NUM_SUBCORES * SUP0 + s * SUP1).astype(jnp.int32)

        pltpu.sync_copy(zeros_hbm.at[pl.ds(s * ROWS_PER_TILE, ROWS_PER_TILE)],
                        acc.at[pl.ds(s * ROWS_PER_TILE, ROWS_PER_TILE)])

        @pl.when(s == NUM_SUBCORES - 1)
        def _zero_tail():
            base = NUM_SUBCORES * ROWS_PER_TILE
            pltpu.sync_copy(zeros_hbm.at[pl.ds(base, ROWS_TAIL)],
                            acc.at[pl.ds(base, ROWS_TAIL)])

        plsc.subcore_barrier()

        def idx_start(p, psel):
            blk = blk_base + p
            pltpu.async_copy(col_hbm.at[blk], col_s.at[psel], isem)
            pltpu.async_copy(row_hbm.at[blk], row_s.at[psel], isem)
            pltpu.async_copy(ew_hbm.at[blk], ew_s.at[psel], isem)

        def idx_wait(p, psel):
            blk = blk_base + p
            pltpu.make_async_copy(
                col_hbm.at[blk], col_s.at[psel], isem).wait()
            pltpu.make_async_copy(
                row_hbm.at[blk], row_s.at[psel], isem).wait()
            pltpu.make_async_copy(
                ew_hbm.at[blk], ew_s.at[psel], isem).wait()

        def gather_start(u, psel, j):
            pltpu.async_copy(x_hbm.at[col_s.at[psel, jnp.int32(j)]],
                             gbufs[u], gsems[u])

        def gather_wait(u, psel, j):
            pltpu.make_async_copy(x_hbm.at[col_s.at[psel, jnp.int32(j)]],
                                  gbufs[u], gsems[u]).wait()

        def scatter_start(u, psel, j):
            pltpu.async_copy(sbufs[u], acc.at[row_s.at[psel, jnp.int32(j)]],
                             ssems[u], add=True)

        def scatter_wait(u, psel, j):
            pltpu.make_async_copy(sbufs[u],
                                  acc.at[row_s.at[psel, jnp.int32(j)]],
                                  ssems[u]).wait()

        def scale(u, psel, j):
            gbuf, sbuf = gbufs[u], sbufs[u]
            j = jnp.int32(j)

            def group_body(g, c2):
                w16 = ew_s[psel, j, pl.ds(g * 16, 16)]
                for jj in range(16):
                    wj = w16[jj]
                    e = g * 16 + jj
                    for t in range(D // 32):
                        ab32 = gbuf[e, pl.ds(t * 16, 16)]
                        ab = plsc.bitcast(ab32, jnp.bfloat16)
                        lo, hi = plsc.unpack(
                            ab, format=plsc.PackFormat.INTERLEAVED)
                        sbuf[e, pl.ds(t * 32, 16)] = lo * wj
                        sbuf[e, pl.ds(t * 32 + 16, 16)] = hi * wj
                return c2

            lax.fori_loop(jnp.int32(0), jnp.int32(CHUNK // 16), group_body,
                          jnp.int32(0), unroll=False)

        idx_start(jnp.int32(0), jnp.int32(0))

        def super_body(p, carry):
            psel = lax.rem(p, jnp.int32(2))
            prev = jnp.int32(1) - psel

            idx_wait(p, psel)
            gather_start(0, psel, 0)

            gather_wait(0, psel, 0)
            gather_start(1, psel, 1)
            scale(0, psel, 0)

            @pl.when(p >= 1)
            def _retire_prev_block():
                scatter_wait(1, prev, SUPER - 1)

            scatter_start(0, psel, 0)

            @pl.when(p + 1 < nsup)
            def _prefetch_next_block():
                idx_start(p + 1, prev)

            def pair_body(t7, c2):
                for par in range(2):
                    j = 1 + t7 * 2 + par
                    u = (1 + par) % 2
                    gather_wait(u, psel, j)
                    gather_start((u + 1) % 2, psel, j + 1)
                    scale(u, psel, j)
                    scatter_wait((u + 1) % 2, psel, j - 1)
                    scatter_start(u, psel, j)
                return c2

            lax.fori_loop(jnp.int32(0), jnp.int32(7), pair_body,
                          jnp.int32(0), unroll=False)

            gather_wait(1, psel, SUPER - 1)
            scale(1, psel, SUPER - 1)
            scatter_wait(0, psel, SUPER - 2)
            scatter_start(1, psel, SUPER - 1)
            return carry

        lax.fori_loop(jnp.int32(0), nsup, super_body,
                      jnp.int32(0), unroll=False)

        scatter_wait(1, lax.rem(nsup - 1, jnp.int32(2)), SUPER - 1)

        plsc.subcore_barrier()
        pltpu.sync_copy(acc.at[pl.ds(s * ROWS_PER_TILE, ROWS_PER_TILE)],
                        out_hbm.at[c, pl.ds(s * ROWS_PER_TILE, ROWS_PER_TILE)])

        @pl.when(s == NUM_SUBCORES - 1)
        def _export_tail():
            base = NUM_SUBCORES * ROWS_PER_TILE
            pltpu.sync_copy(acc.at[pl.ds(base, ROWS_TAIL)],
                            out_hbm.at[c, pl.ds(base, ROWS_TAIL)])

    return agg(xh, col3, row3, ew3, zeros)


def _tc_linear(partials, Wt, b2):
    BM = 1000
    grid = (N_NODES // BM,)

    def body(p_ref, wt_ref, b_ref, o_ref):
        acc = p_ref[0] + p_ref[1]
        o_ref[...] = (
            jnp.dot(acc, wt_ref[...], preferred_element_type=jnp.float32)
            + b_ref[...]
        )

    return pl.pallas_call(
        body,
        grid=grid,
        in_specs=[
            pl.BlockSpec((NUM_CORES, BM, D), lambda i: (i * 0, i, i * 0)),
            pl.BlockSpec((D, D), lambda i: (i * 0, i * 0)),
            pl.BlockSpec((1, D), lambda i: (i * 0, i * 0)),
        ],
        out_specs=pl.BlockSpec((BM, D), lambda i: (i, i * 0)),
        out_shape=jax.ShapeDtypeStruct((N_NODES, D), jnp.float32),
    )(partials, Wt, b2)


def kernel(x, edge_index, edge_weight, W, b):
    x = x.astype(jnp.float32)
    row = edge_index[0].astype(jnp.int32)
    col = edge_index[1].astype(jnp.int32)
    ew = edge_weight.astype(jnp.float32)

    xh = lax.bitcast_convert_type(
        x[:, _Q].astype(jnp.bfloat16).reshape(N_NODES, D // 2, 2), jnp.int32)

    pad = E_PAD - N_EDGES
    row3 = jnp.concatenate([row, jnp.zeros((pad,), jnp.int32)]).reshape(
        N_BLOCKS, SUPER, CHUNK)
    col3 = jnp.concatenate([col, jnp.zeros((pad,), jnp.int32)]).reshape(
        N_BLOCKS, SUPER, CHUNK)
    ew3 = jnp.concatenate([ew, jnp.zeros((pad,), jnp.float32)]).reshape(
        N_BLOCKS, SUPER, CHUNK)
    zeros = jnp.zeros((N_NODES, D), jnp.float32)

    partials = _sc_aggregate(xh, col3, row3, ew3, zeros)

    Wt = W.astype(jnp.float32).T
    b2 = b.astype(jnp.float32).reshape(1, D)
    return _tc_linear(partials, Wt, b2)


# speedup vs baseline: 2.3474x; 2.3474x over previous
import functools

import jax
import jax.numpy as jnp
from jax import lax
from jax.experimental import pallas as pl
from jax.experimental.pallas import tpu as pltpu
from jax.experimental.pallas import tpu_sc as plsc

N_NODES = 10000
D = 128
N_EDGES = 320000

NUM_CORES = 2
NUM_SUBCORES = 16
NUM_WORKERS = NUM_CORES * NUM_SUBCORES
CHUNK = 96
SUPER = 15
SUP0 = 9
SUP1 = 5
N_BLOCKS = NUM_SUBCORES * (SUP0 + SUP1)
E_PAD = N_BLOCKS * SUPER * CHUNK
NBUF = 3
ROWS_PER_TILE = 624
ROWS_TAIL = N_NODES - NUM_SUBCORES * ROWS_PER_TILE


def _sc_aggregate(x, col3, row3, ew3, zeros):
    mesh = plsc.VectorSubcoreMesh(core_axis_name="c", subcore_axis_name="s")

    @functools.partial(
        pl.kernel,
        mesh=mesh,
        out_type=jax.ShapeDtypeStruct((NUM_CORES, N_NODES, D), jnp.float32),
        scratch_types=[
            pltpu.VMEM((2, SUPER, CHUNK), jnp.int32),
            pltpu.VMEM((2, SUPER, CHUNK), jnp.int32),
            pltpu.VMEM((2, SUPER, CHUNK), jnp.float32),
            pltpu.VMEM((CHUNK, D), jnp.float32),
            pltpu.VMEM((CHUNK, D), jnp.float32),
            pltpu.VMEM((CHUNK, D), jnp.float32),
            pltpu.VMEM_SHARED((N_NODES, D), jnp.float32),
            pltpu.SemaphoreType.DMA,
            pltpu.SemaphoreType.DMA,
            pltpu.SemaphoreType.DMA,
            pltpu.SemaphoreType.DMA,
            pltpu.SemaphoreType.DMA,
            pltpu.SemaphoreType.DMA,
            pltpu.SemaphoreType.DMA,
        ],
    )
    def agg(x_hbm, col_hbm, row_hbm, ew_hbm, zeros_hbm, out_hbm,
            col_s, row_s, ew_s, d0, d1, d2, acc,
            g0, g1, g2, s0, s1, s2, isem):
        c = lax.axis_index("c")
        s = lax.axis_index("s")
        bufs = (d0, d1, d2)
        gsems = (g0, g1, g2)
        ssems = (s0, s1, s2)

        nsup = jnp.where(c == 0, jnp.int32(SUP0), jnp.int32(SUP1))
        blk_base = jnp.where(c == 0, s * SUP0,
                             NUM_SUBCORES * SUP0 + s * SUP1).astype(jnp.int32)

        pltpu.sync_copy(zeros_hbm.at[pl.ds(s * ROWS_PER_TILE, ROWS_PER_TILE)],
                        acc.at[pl.ds(s * ROWS_PER_TILE, ROWS_PER_TILE)])

        @pl.when(s == NUM_SUBCORES - 1)
        def _zero_tail():
            base = NUM_SUBCORES * ROWS_PER_TILE
            pltpu.sync_copy(zeros_hbm.at[pl.ds(base, ROWS_TAIL)],
                            acc.at[pl.ds(base, ROWS_TAIL)])

        plsc.subcore_barrier()

        def idx_start(p, psel):
            blk = blk_base + p
            pltpu.async_copy(col_hbm.at[blk], col_s.at[psel], isem)
            pltpu.async_copy(row_hbm.at[blk], row_s.at[psel], isem)
            pltpu.async_copy(ew_hbm.at[blk], ew_s.at[psel], isem)

        def idx_wait(p, psel):
            blk = blk_base + p
            pltpu.make_async_copy(
                col_hbm.at[blk], col_s.at[psel], isem).wait()
            pltpu.make_async_copy(
                row_hbm.at[blk], row_s.at[psel], isem).wait()
            pltpu.make_async_copy(
                ew_hbm.at[blk], ew_s.at[psel], isem).wait()

        def gather_start(u, psel, j):
            pltpu.async_copy(x_hbm.at[col_s.at[psel, jnp.int32(j)]],
                             bufs[u], gsems[u])

        def gather_wait(u, psel, j):
            pltpu.make_async_copy(x_hbm.at[col_s.at[psel, jnp.int32(j)]],
                                  bufs[u], gsems[u]).wait()

        def scatter_start(u, psel, j):
            pltpu.async_copy(bufs[u], acc.at[row_s.at[psel, jnp.int32(j)]],
                             ssems[u], add=True)

        def scatter_wait(u, psel, j):
            pltpu.make_async_copy(bufs[u],
                                  acc.at[row_s.at[psel, jnp.int32(j)]],
                                  ssems[u]).wait()

        def scale(u, psel, j):
            buf = bufs[u]
            j = jnp.int32(j)

            def group_body(g, c2):
                w16 = ew_s[psel, j, pl.ds(g * 16, 16)]
                for jj in range(16):
                    wj = w16[jj]
                    for t in range(D // 16):
                        sl = pl.ds(t * 16, 16)
                        buf[g * 16 + jj, sl] = buf[g * 16 + jj, sl] * wj
                return c2

            lax.fori_loop(jnp.int32(0), jnp.int32(CHUNK // 16), group_body,
                          jnp.int32(0), unroll=False)

        idx_start(jnp.int32(0), jnp.int32(0))

        def super_body(p, carry):
            psel = lax.rem(p, jnp.int32(2))
            prev = jnp.int32(1) - psel

            idx_wait(p, psel)
            gather_start(0, psel, 0)
            gather_start(1, psel, 1)

            gather_wait(0, psel, 0)
            scale(0, psel, 0)

            @pl.when(p >= 1)
            def _retire_prev_block():
                scatter_wait(2, prev, SUPER - 1)

            scatter_start(0, psel, 0)
            gather_start(2, psel, 2)

            gather_wait(1, psel, 1)
            scale(1, psel, 1)
            scatter_wait(0, psel, 0)
            scatter_start(1, psel, 1)
            gather_start(0, psel, 3)

            gather_wait(2, psel, 2)
            scale(2, psel, 2)
            scatter_wait(1, psel, 1)
            scatter_start(2, psel, 2)
            gather_start(1, psel, 4)

            @pl.when(p + 1 < nsup)
            def _prefetch_next_block():
                idx_start(p + 1, prev)

            def tri_body(t5, c2):
                for u in range(3):
                    j = t5 * 3 + u
                    gather_wait(u, psel, j)
                    scale(u, psel, j)
                    scatter_wait((u + 2) % 3, psel, j - 1)
                    scatter_start(u, psel, j)

                    @pl.when(j + 2 < SUPER)
                    def _next_gather(u=u, j=j):
                        gather_start((u + 2) % 3, psel, j + 2)
                return c2

            lax.fori_loop(jnp.int32(1), jnp.int32(SUPER // 3), tri_body,
                          jnp.int32(0), unroll=False)
            return carry

        lax.fori_loop(jnp.int32(0), nsup, super_body,
                      jnp.int32(0), unroll=False)

        scatter_wait((SUPER - 1) % NBUF, lax.rem(nsup - 1, jnp.int32(2)),
                     SUPER - 1)

        plsc.subcore_barrier()
        pltpu.sync_copy(acc.at[pl.ds(s * ROWS_PER_TILE, ROWS_PER_TILE)],
                        out_hbm.at[c, pl.ds(s * ROWS_PER_TILE, ROWS_PER_TILE)])

        @pl.when(s == NUM_SUBCORES - 1)
        def _export_tail():
            base = NUM_SUBCORES * ROWS_PER_TILE
            pltpu.sync_copy(acc.at[pl.ds(base, ROWS_TAIL)],
                            out_hbm.at[c, pl.ds(base, ROWS_TAIL)])

    return agg(x, col3, row3, ew3, zeros)


def _tc_linear(partials, Wt, b2):
    BM = 1000
    grid = (N_NODES // BM,)

    def body(p_ref, wt_ref, b_ref, o_ref):
        acc = p_ref[0] + p_ref[1]
        o_ref[...] = (
            jnp.dot(acc, wt_ref[...], preferred_element_type=jnp.float32)
            + b_ref[...]
        )

    return pl.pallas_call(
        body,
        grid=grid,
        in_specs=[
            pl.BlockSpec((NUM_CORES, BM, D), lambda i: (i * 0, i, i * 0)),
            pl.BlockSpec((D, D), lambda i: (i * 0, i * 0)),
            pl.BlockSpec((1, D), lambda i: (i * 0, i * 0)),
        ],
        out_specs=pl.BlockSpec((BM, D), lambda i: (i, i * 0)),
        out_shape=jax.ShapeDtypeStruct((N_NODES, D), jnp.float32),
    )(partials, Wt, b2)


def kernel(x, edge_index, edge_weight, W, b):
    x = x.astype(jnp.float32)
    row = edge_index[0].astype(jnp.int32)
    col = edge_index[1].astype(jnp.int32)
    ew = edge_weight.astype(jnp.float32)

    pad = E_PAD - N_EDGES
    row3 = jnp.concatenate([row, jnp.zeros((pad,), jnp.int32)]).reshape(
        N_BLOCKS, SUPER, CHUNK)
    col3 = jnp.concatenate([col, jnp.zeros((pad,), jnp.int32)]).reshape(
        N_BLOCKS, SUPER, CHUNK)
    ew3 = jnp.concatenate([ew, jnp.zeros((pad,), jnp.float32)]).reshape(
        N_BLOCKS, SUPER, CHUNK)
    zeros = jnp.zeros((N_NODES, D), jnp.float32)

    partials = _sc_aggregate(x, col3, row3, ew3, zeros)

    Wt = W.astype(jnp.float32).T
    b2 = b.astype(jnp.float32).reshape(1, D)
    return _tc_linear(partials, Wt, b2)
